# baseline (speedup 1.0000x reference)
import functools

import jax
import jax.numpy as jnp
from jax import lax
from jax.experimental import pallas as pl
from jax.experimental.pallas import tpu as pltpu
from jax.experimental.pallas import tpu_sc as plsc

_KC = 30
_KP = 32


def _stage1a_body(x_ref, aW_ref, ab_ref, S_ref):
    x = x_ref[...]
    logits = jnp.dot(x, aW_ref[...], preferred_element_type=jnp.float32)
    logits = logits + ab_ref[...]
    kmask = lax.broadcasted_iota(jnp.int32, logits.shape, 1) < _KC
    logits = jnp.where(kmask, logits, -1e30)
    m = jnp.max(logits, axis=1, keepdims=True)
    e = jnp.exp(logits - m)
    S_ref[...] = e / jnp.sum(e, axis=1, keepdims=True)


def _stage1b_body(x_ref, pW_ref, pb_ref, S_ref, Z_ref, SS_ref):
    i = pl.program_id(0)
    x = x_ref[...]
    S = S_ref[...]
    Xp = jnp.dot(x, pW_ref[...], preferred_element_type=jnp.float32) + pb_ref[...]
    Zp = lax.dot_general(S, Xp, (((0,), (0,)), ((), ())),
                         preferred_element_type=jnp.float32)
    SSp = lax.dot_general(S, S, (((0,), (0,)), ((), ())),
                          preferred_element_type=jnp.float32)

    @pl.when(i == 0)
    def _init():
        Z_ref[...] = Zp
        SS_ref[...] = SSp

    @pl.when(i != 0)
    def _acc():
        Z_ref[...] += Zp
        SS_ref[...] += SSp


_CH = 512
_Q = _CH // 4


def _make_gather(E_pad, NT):
    info = plsc.get_sparse_core_info()
    nw = info.num_cores * info.num_subcores
    ns = info.num_subcores
    tpw = (E_pad // _CH) // nw
    rpt = NT // ns
    mesh = plsc.VectorSubcoreMesh(core_axis_name="c", subcore_axis_name="s")

    @functools.partial(
        pl.kernel,
        mesh=mesh,
        compiler_params=pltpu.CompilerParams(use_tc_tiling_on_sc=False),
        out_type=(jax.ShapeDtypeStruct((E_pad // 4, 128), jnp.float32),
                  jax.ShapeDtypeStruct((E_pad // 4, 128), jnp.float32)),
        scratch_types=[
            [pltpu.VMEM((_CH,), jnp.int32) for _ in range(4)],
            [pltpu.VMEM((_CH, _KP), jnp.float32) for _ in range(4)],
            pltpu.VMEM_SHARED((NT, _KP), jnp.float32),
            [pltpu.SemaphoreType.DMA for _ in range(4)],
            [pltpu.SemaphoreType.DMA for _ in range(4)],
            [pltpu.SemaphoreType.DMA for _ in range(4)],
        ],
    )
    def gather_k(table, row_idx, col_idx, out_r, out_c,
                 idxb, bufb, stab, si, sg, sw):
        sid = lax.axis_index("s")
        wid = sid * info.num_cores + lax.axis_index("c")
        o = 0
        while o < rpt:
            n = min(_CH, rpt - o)
            pltpu.sync_copy(table.at[pl.ds(sid * rpt + o, n)],
                            bufb[0].at[pl.ds(0, n)])
            pltpu.sync_copy(bufb[0].at[pl.ds(0, n)],
                            stab.at[pl.ds(sid * rpt + o, n)])
            o += n
        plsc.subcore_barrier()
        jobs = []
        for t in range(tpw):
            cid = t * nw + wid
            for idx_hbm, out_hbm in ((row_idx, out_r), (col_idx, out_c)):
                jobs.append((idx_hbm, out_hbm, cid * _CH, cid * _Q))
        nj = len(jobs)

        def start_idx(j):
            src, _, eoff, _ = jobs[j]
            return pltpu.async_copy(src.at[pl.ds(eoff, _CH)], idxb[j % 4],
                                    si[j % 4])

        def start_gather(j):
            return pltpu.async_copy(stab.at[idxb[j % 4]], bufb[j % 4],
                                    sg[j % 4])

        def start_writes(j):
            _, out, _, poff = jobs[j]
            buf = bufb[j % 4]
            hs = []
            for q in range(4):
                hs.append(pltpu.async_copy(
                    buf.at[pl.ds(q * _Q, _Q)],
                    out.at[pl.ds(poff, _Q), pl.ds(q * _KP, _KP)],
                    sw[j % 4]))
            return hs

        hi, hg, hw = {}, {}, {}
        for j in range(min(4, nj)):
            hi[j] = start_idx(j)
        for j in range(min(2, nj)):
            hi[j].wait()
            hg[j] = start_gather(j)
        for j in range(nj):
            hg[j].wait()
            if j + 2 < nj:
                hi[j + 2].wait()
                if j >= 2:
                    for h in hw[j - 2]:
                        h.wait()
                hg[j + 2] = start_gather(j + 2)
            hw[j] = start_writes(j)
            if j + 4 < nj:
                hi[j + 4] = start_idx(j + 4)
        for j in range(max(0, nj - 4), nj):
            for h in hw[j]:
                h.wait()

    return gather_k


def _stage3_body(sr_ref, sc_ref, Z_ref, SS_ref,
                 m1a_ref, m1b_ref, m1b1_ref, m1W2_ref, m1b2_ref, lin1_ref,
                 m2a_ref, m2b_ref, m2b1_ref, m2W2_ref, m2b2_ref, lin2_ref,
                 finW_ref, finb_ref,
                 out_ref, mc_ref, ol_ref,
                 adj_ref, vol_ref):
    i = pl.program_id(0)
    sr = sr_ref[...]
    sc = sc_ref[...]
    adj_p = None
    for q in range(4):
        p = lax.dot_general(sr[:, q * _KP:(q + 1) * _KP],
                            sc[:, q * _KP:(q + 1) * _KP],
                            (((0,), (0,)), ((), ())),
                            preferred_element_type=jnp.float32)
        adj_p = p if adj_p is None else adj_p + p

    @pl.when(i == 0)
    def _init():
        adj_ref[...] = adj_p
        vol_ref[0, 0] = jnp.sum(sr)

    @pl.when(i != 0)
    def _acc():
        adj_ref[...] += adj_p
        vol_ref[0, 0] += jnp.sum(sr)

    @pl.when(i == pl.num_programs(0) - 1)
    def _finale():
        f32 = jnp.float32
        adj = adj_ref[...]
        vol = vol_ref[0, 0]
        rio = lax.broadcasted_iota(jnp.int32, (_KP, _KP), 0)
        cio = lax.broadcasted_iota(jnp.int32, (_KP, _KP), 1)
        eye30 = jnp.where((rio == cio) & (rio < _KC), 1.0, 0.0).astype(f32)
        cut = jnp.sum(adj * eye30)
        mc_ref[...] = jnp.broadcast_to(-cut / (vol + 1e-9), (1, 1))
        d = SS_ref[...] - eye30
        ol_ref[...] = jnp.broadcast_to(jnp.sqrt(jnp.sum(d * d)), (1, 1))

        emask = (adj > 0).astype(f32)
        e2 = _KP * _KP
        eio = lax.broadcasted_iota(jnp.int32, (e2, _KP), 0)
        aio = lax.broadcasted_iota(jnp.int32, (e2, _KP), 1)
        R = (eio // _KP == aio).astype(f32)
        T = (eio % _KP == aio).astype(f32)
        emask_vec = jnp.sum(
            jnp.dot(R, emask, preferred_element_type=f32) * T,
            axis=1, keepdims=True)

        def pgagg(zx, w1a, w1b, b1, w2, b2, lin_w):
            a = jnp.dot(zx, w1a, preferred_element_type=f32)
            b = jnp.dot(zx, w1b, preferred_element_type=f32)
            h = jnp.maximum(
                jnp.dot(R, a, preferred_element_type=f32)
                + jnp.dot(T, b, preferred_element_type=f32) + b1, 0.0)
            s = jnp.dot(h, w2, preferred_element_type=f32) + b2
            mvec = jax.nn.sigmoid(s) * emask_vec
            zc = jnp.dot(T, zx, preferred_element_type=f32)
            msg = zc * mvec
            agg = lax.dot_general(R, msg, (((0,), (0,)), ((), ())),
                                  preferred_element_type=f32)
            norm = lax.dot_general(R, mvec, (((0,), (0,)), ((), ())),
                                   preferred_element_type=f32)
            combined = agg / (norm + 1e-9) + zx
            return jnp.maximum(
                jnp.dot(combined, lin_w, preferred_element_type=f32), 0.0)

        zx = Z_ref[...]
        h1 = pgagg(zx, m1a_ref[...], m1b_ref[...], m1b1_ref[...],
                   m1W2_ref[...], m1b2_ref[...], lin1_ref[...])
        h2 = pgagg(h1, m2a_ref[...], m2b_ref[...], m2b1_ref[...],
                   m2W2_ref[...], m2b2_ref[...], lin2_ref[...])
        outv = jnp.dot(h2, finW_ref[...], preferred_element_type=f32)
        outv = outv + finb_ref[...]
        out_ref[...] = outv[:_KC, :]


def kernel(x, edge_index, assign_W, assign_b, proj_W, proj_b,
           m1_W1, m1_b1, m1_W2, m1_b2, lin1_W,
           m2_W1, m2_b1, m2_W2, m2_b2, lin2_W, fin_W, fin_b):
    f32 = jnp.float32
    N, Din = x.shape
    E = edge_index.shape[1]
    H = lin1_W.shape[1]
    Dout = fin_W.shape[1]

    aW = jnp.pad(assign_W, ((0, 0), (0, _KP - _KC)))
    ab = jnp.pad(assign_b, (0, _KP - _KC)).reshape(1, _KP)
    pb = proj_b.reshape(1, Din)

    bn = 2000
    while N % bn != 0:
        bn //= 2
    grid1 = N // bn
    S32 = pl.pallas_call(
        _stage1a_body,
        grid=(grid1,),
        in_specs=[
            pl.BlockSpec((bn, Din), lambda i: (i, 0)),
            pl.BlockSpec((Din, _KP), lambda i: (0, 0)),
            pl.BlockSpec((1, _KP), lambda i: (0, 0)),
        ],
        out_specs=pl.BlockSpec((bn, _KP), lambda i: (i, 0)),
        out_shape=jax.ShapeDtypeStruct((N, _KP), f32),
    )(x, aW, ab)

    unit = 32 * _CH
    E_pad = ((E + unit - 1) // unit) * unit
    NT = ((N + 8 + 127) // 128) * 128
    table = jnp.pad(S32, ((0, NT - N), (0, 0)))
    row = jnp.pad(edge_index[0].astype(jnp.int32), (0, E_pad - E),
                  constant_values=N)
    col = jnp.pad(edge_index[1].astype(jnp.int32), (0, E_pad - E),
                  constant_values=N)
    Srow, Scol = _make_gather(E_pad, NT)(table, row, col)

    Z_pad, SS = pl.pallas_call(
        _stage1b_body,
        grid=(grid1,),
        in_specs=[
            pl.BlockSpec((bn, Din), lambda i: (i, 0)),
            pl.BlockSpec((Din, Din), lambda i: (0, 0)),
            pl.BlockSpec((1, Din), lambda i: (0, 0)),
            pl.BlockSpec((bn, _KP), lambda i: (i, 0)),
        ],
        out_specs=[
            pl.BlockSpec((_KP, Din), lambda i: (0, 0)),
            pl.BlockSpec((_KP, _KP), lambda i: (0, 0)),
        ],
        out_shape=[
            jax.ShapeDtypeStruct((_KP, Din), f32),
            jax.ShapeDtypeStruct((_KP, _KP), f32),
        ],
    )(x, proj_W, pb, S32)

    np4 = E_pad // 4
    be = 2048
    while np4 % be != 0:
        be //= 2
    grid3 = np4 // be
    const = lambda i: (0, 0)
    wspecs = [
        pl.BlockSpec((Din, 64), const), pl.BlockSpec((Din, 64), const),
        pl.BlockSpec((1, 64), const), pl.BlockSpec((64, 1), const),
        pl.BlockSpec((1, 1), const), pl.BlockSpec((Din, H), const),
        pl.BlockSpec((H, 64), const), pl.BlockSpec((H, 64), const),
        pl.BlockSpec((1, 64), const), pl.BlockSpec((64, 1), const),
        pl.BlockSpec((1, 1), const), pl.BlockSpec((H, H), const),
        pl.BlockSpec((H, Dout), const), pl.BlockSpec((1, Dout), const),
    ]
    out, mc, ol = pl.pallas_call(
        _stage3_body,
        grid=(grid3,),
        in_specs=[
            pl.BlockSpec((be, 128), lambda i: (i, 0)),
            pl.BlockSpec((be, 128), lambda i: (i, 0)),
            pl.BlockSpec((_KP, Din), const),
            pl.BlockSpec((_KP, _KP), const),
        ] + wspecs,
        out_specs=[
            pl.BlockSpec((_KC, Dout), const),
            pl.BlockSpec((1, 1), const),
            pl.BlockSpec((1, 1), const),
        ],
        out_shape=[
            jax.ShapeDtypeStruct((_KC, Dout), f32),
            jax.ShapeDtypeStruct((1, 1), f32),
            jax.ShapeDtypeStruct((1, 1), f32),
        ],
        scratch_shapes=[
            pltpu.VMEM((_KP, _KP), f32),
            pltpu.SMEM((1, 1), f32),
        ],
    )(Srow, Scol, Z_pad, SS,
      m1_W1[:Din], m1_W1[Din:], m1_b1.reshape(1, 64),
      m1_W2, m1_b2.reshape(1, 1), lin1_W,
      m2_W1[:H], m2_W1[H:], m2_b1.reshape(1, 64),
      m2_W2, m2_b2.reshape(1, 1), lin2_W,
      fin_W, fin_b.reshape(1, Dout))

    return (out, mc[0, 0], ol[0, 0], Z_pad[:_KC], S32[:, :_KC])


# speedup vs baseline: 1.0492x; 1.0492x over previous
import functools

import jax
import jax.numpy as jnp
from jax import lax
from jax.experimental import pallas as pl
from jax.experimental.pallas import tpu as pltpu
from jax.experimental.pallas import tpu_sc as plsc

_KC = 30
_KP = 32


def _stage1_body(x_ref, aW_ref, ab_ref, pW_ref, pb_ref, S_ref, Z_ref, SS_ref):
    i = pl.program_id(0)
    x = x_ref[...]
    logits = jnp.dot(x, aW_ref[...], preferred_element_type=jnp.float32)
    logits = logits + ab_ref[...]
    kmask = lax.broadcasted_iota(jnp.int32, logits.shape, 1) < _KC
    logits = jnp.where(kmask, logits, -1e30)
    m = jnp.max(logits, axis=1, keepdims=True)
    e = jnp.exp(logits - m)
    S = e / jnp.sum(e, axis=1, keepdims=True)
    S_ref[...] = S
    Xp = jnp.dot(x, pW_ref[...], preferred_element_type=jnp.float32) + pb_ref[...]
    Zp = lax.dot_general(S, Xp, (((0,), (0,)), ((), ())),
                         preferred_element_type=jnp.float32)
    SSp = lax.dot_general(S, S, (((0,), (0,)), ((), ())),
                          preferred_element_type=jnp.float32)

    @pl.when(i == 0)
    def _init():
        Z_ref[...] = Zp
        SS_ref[...] = SSp

    @pl.when(i != 0)
    def _acc():
        Z_ref[...] += Zp
        SS_ref[...] += SSp


_CH = 512
_Q = _CH // 4


def _make_gather(E_pad, NT):
    info = plsc.get_sparse_core_info()
    nw = info.num_cores * info.num_subcores
    ns = info.num_subcores
    tpw = (E_pad // _CH) // nw
    rpt = NT // ns
    mesh = plsc.VectorSubcoreMesh(core_axis_name="c", subcore_axis_name="s")

    @functools.partial(
        pl.kernel,
        mesh=mesh,
        compiler_params=pltpu.CompilerParams(use_tc_tiling_on_sc=False),
        out_type=(jax.ShapeDtypeStruct((E_pad // 4, 128), jnp.float32),
                  jax.ShapeDtypeStruct((E_pad // 4, 128), jnp.float32)),
        scratch_types=[
            [pltpu.VMEM((_CH,), jnp.int32) for _ in range(4)],
            [pltpu.VMEM((_CH, _KP), jnp.float32) for _ in range(4)],
            pltpu.VMEM_SHARED((NT, _KP), jnp.float32),
            [pltpu.SemaphoreType.DMA for _ in range(4)],
            [pltpu.SemaphoreType.DMA for _ in range(4)],
            [pltpu.SemaphoreType.DMA for _ in range(4)],
        ],
    )
    def gather_k(table, row_idx, col_idx, out_r, out_c,
                 idxb, bufb, stab, si, sg, sw):
        sid = lax.axis_index("s")
        wid = sid * info.num_cores + lax.axis_index("c")
        o = 0
        while o < rpt:
            n = min(_CH, rpt - o)
            pltpu.sync_copy(table.at[pl.ds(sid * rpt + o, n)],
                            bufb[0].at[pl.ds(0, n)])
            pltpu.sync_copy(bufb[0].at[pl.ds(0, n)],
                            stab.at[pl.ds(sid * rpt + o, n)])
            o += n
        plsc.subcore_barrier()
        jobs = []
        for t in range(tpw):
            cid = t * nw + wid
            for idx_hbm, out_hbm in ((row_idx, out_r), (col_idx, out_c)):
                jobs.append((idx_hbm, out_hbm, cid * _CH, cid * _Q))
        nj = len(jobs)

        def start_idx(j):
            src, _, eoff, _ = jobs[j]
            return pltpu.async_copy(src.at[pl.ds(eoff, _CH)], idxb[j % 4],
                                    si[j % 4])

        def start_gather(j):
            return pltpu.async_copy(stab.at[idxb[j % 4]], bufb[j % 4],
                                    sg[j % 4])

        def start_writes(j):
            _, out, _, poff = jobs[j]
            buf = bufb[j % 4]
            hs = []
            for q in range(4):
                hs.append(pltpu.async_copy(
                    buf.at[pl.ds(q * _Q, _Q)],
                    out.at[pl.ds(poff, _Q), pl.ds(q * _KP, _KP)],
                    sw[j % 4]))
            return hs

        hi, hg, hw = {}, {}, {}
        for j in range(min(4, nj)):
            hi[j] = start_idx(j)
        for j in range(min(2, nj)):
            hi[j].wait()
            hg[j] = start_gather(j)
        for j in range(nj):
            hg[j].wait()
            if j + 2 < nj:
                hi[j + 2].wait()
                if j >= 2:
                    for h in hw[j - 2]:
                        h.wait()
                hg[j + 2] = start_gather(j + 2)
            hw[j] = start_writes(j)
            if j + 4 < nj:
                hi[j + 4] = start_idx(j + 4)
        for j in range(max(0, nj - 4), nj):
            for h in hw[j]:
                h.wait()

    return gather_k


def _stage3_body(sr_ref, sc_ref, Z_ref, SS_ref,
                 m1a_ref, m1b_ref, m1b1_ref, m1W2_ref, m1b2_ref, lin1_ref,
                 m2a_ref, m2b_ref, m2b1_ref, m2W2_ref, m2b2_ref, lin2_ref,
                 finW_ref, finb_ref,
                 out_ref, mc_ref, ol_ref,
                 adj_ref, vol_ref):
    i = pl.program_id(0)
    sr = sr_ref[...]
    sc = sc_ref[...]
    adj_p = None
    for q in range(4):
        p = lax.dot_general(sr[:, q * _KP:(q + 1) * _KP],
                            sc[:, q * _KP:(q + 1) * _KP],
                            (((0,), (0,)), ((), ())),
                            preferred_element_type=jnp.float32)
        adj_p = p if adj_p is None else adj_p + p

    @pl.when(i == 0)
    def _init():
        adj_ref[...] = adj_p
        vol_ref[0, 0] = jnp.sum(sr)

    @pl.when(i != 0)
    def _acc():
        adj_ref[...] += adj_p
        vol_ref[0, 0] += jnp.sum(sr)

    @pl.when(i == pl.num_programs(0) - 1)
    def _finale():
        f32 = jnp.float32
        adj = adj_ref[...]
        vol = vol_ref[0, 0]
        rio = lax.broadcasted_iota(jnp.int32, (_KP, _KP), 0)
        cio = lax.broadcasted_iota(jnp.int32, (_KP, _KP), 1)
        eye30 = jnp.where((rio == cio) & (rio < _KC), 1.0, 0.0).astype(f32)
        cut = jnp.sum(adj * eye30)
        mc_ref[...] = jnp.broadcast_to(-cut / (vol + 1e-9), (1, 1))
        d = SS_ref[...] - eye30
        ol_ref[...] = jnp.broadcast_to(jnp.sqrt(jnp.sum(d * d)), (1, 1))

        emask = (adj > 0).astype(f32)
        e2 = _KP * _KP
        eio = lax.broadcasted_iota(jnp.int32, (e2, _KP), 0)
        aio = lax.broadcasted_iota(jnp.int32, (e2, _KP), 1)
        R = (eio // _KP == aio).astype(f32)
        T = (eio % _KP == aio).astype(f32)
        emask_vec = jnp.sum(
            jnp.dot(R, emask, preferred_element_type=f32) * T,
            axis=1, keepdims=True)

        def pgagg(zx, w1a, w1b, b1, w2, b2, lin_w):
            a = jnp.dot(zx, w1a, preferred_element_type=f32)
            b = jnp.dot(zx, w1b, preferred_element_type=f32)
            h = jnp.maximum(
                jnp.dot(R, a, preferred_element_type=f32)
                + jnp.dot(T, b, preferred_element_type=f32) + b1, 0.0)
            s = jnp.dot(h, w2, preferred_element_type=f32) + b2
            mvec = jax.nn.sigmoid(s) * emask_vec
            zc = jnp.dot(T, zx, preferred_element_type=f32)
            msg = zc * mvec
            agg = lax.dot_general(R, msg, (((0,), (0,)), ((), ())),
                                  preferred_element_type=f32)
            norm = lax.dot_general(R, mvec, (((0,), (0,)), ((), ())),
                                   preferred_element_type=f32)
            combined = agg / (norm + 1e-9) + zx
            return jnp.maximum(
                jnp.dot(combined, lin_w, preferred_element_type=f32), 0.0)

        zx = Z_ref[...]
        h1 = pgagg(zx, m1a_ref[...], m1b_ref[...], m1b1_ref[...],
                   m1W2_ref[...], m1b2_ref[...], lin1_ref[...])
        h2 = pgagg(h1, m2a_ref[...], m2b_ref[...], m2b1_ref[...],
                   m2W2_ref[...], m2b2_ref[...], lin2_ref[...])
        outv = jnp.dot(h2, finW_ref[...], preferred_element_type=f32)
        outv = outv + finb_ref[...]
        out_ref[...] = outv[:_KC, :]


def kernel(x, edge_index, assign_W, assign_b, proj_W, proj_b,
           m1_W1, m1_b1, m1_W2, m1_b2, lin1_W,
           m2_W1, m2_b1, m2_W2, m2_b2, lin2_W, fin_W, fin_b):
    f32 = jnp.float32
    N, Din = x.shape
    E = edge_index.shape[1]
    H = lin1_W.shape[1]
    Dout = fin_W.shape[1]

    aW = jnp.pad(assign_W, ((0, 0), (0, _KP - _KC)))
    ab = jnp.pad(assign_b, (0, _KP - _KC)).reshape(1, _KP)
    pb = proj_b.reshape(1, Din)

    bn = 2000
    while N % bn != 0:
        bn //= 2
    grid1 = N // bn
    S32, Z_pad, SS = pl.pallas_call(
        _stage1_body,
        grid=(grid1,),
        in_specs=[
            pl.BlockSpec((bn, Din), lambda i: (i, 0)),
            pl.BlockSpec((Din, _KP), lambda i: (0, 0)),
            pl.BlockSpec((1, _KP), lambda i: (0, 0)),
            pl.BlockSpec((Din, Din), lambda i: (0, 0)),
            pl.BlockSpec((1, Din), lambda i: (0, 0)),
        ],
        out_specs=[
            pl.BlockSpec((bn, _KP), lambda i: (i, 0)),
            pl.BlockSpec((_KP, Din), lambda i: (0, 0)),
            pl.BlockSpec((_KP, _KP), lambda i: (0, 0)),
        ],
        out_shape=[
            jax.ShapeDtypeStruct((N, _KP), f32),
            jax.ShapeDtypeStruct((_KP, Din), f32),
            jax.ShapeDtypeStruct((_KP, _KP), f32),
        ],
    )(x, aW, ab, proj_W, pb)

    unit = 32 * _CH
    E_pad = ((E + unit - 1) // unit) * unit
    NT = ((N + 8 + 127) // 128) * 128
    table = jnp.pad(S32, ((0, NT - N), (0, 0)))
    row = jnp.pad(edge_index[0].astype(jnp.int32), (0, E_pad - E),
                  constant_values=N)
    col = jnp.pad(edge_index[1].astype(jnp.int32), (0, E_pad - E),
                  constant_values=N)
    Srow, Scol = _make_gather(E_pad, NT)(table, row, col)

    np4 = E_pad // 4
    be = 2048
    while np4 % be != 0:
        be //= 2
    grid3 = np4 // be
    const = lambda i: (0, 0)
    wspecs = [
        pl.BlockSpec((Din, 64), const), pl.BlockSpec((Din, 64), const),
        pl.BlockSpec((1, 64), const), pl.BlockSpec((64, 1), const),
        pl.BlockSpec((1, 1), const), pl.BlockSpec((Din, H), const),
        pl.BlockSpec((H, 64), const), pl.BlockSpec((H, 64), const),
        pl.BlockSpec((1, 64), const), pl.BlockSpec((64, 1), const),
        pl.BlockSpec((1, 1), const), pl.BlockSpec((H, H), const),
        pl.BlockSpec((H, Dout), const), pl.BlockSpec((1, Dout), const),
    ]
    out, mc, ol = pl.pallas_call(
        _stage3_body,
        grid=(grid3,),
        in_specs=[
            pl.BlockSpec((be, 128), lambda i: (i, 0)),
            pl.BlockSpec((be, 128), lambda i: (i, 0)),
            pl.BlockSpec((_KP, Din), const),
            pl.BlockSpec((_KP, _KP), const),
        ] + wspecs,
        out_specs=[
            pl.BlockSpec((_KC, Dout), const),
            pl.BlockSpec((1, 1), const),
            pl.BlockSpec((1, 1), const),
        ],
        out_shape=[
            jax.ShapeDtypeStruct((_KC, Dout), f32),
            jax.ShapeDtypeStruct((1, 1), f32),
            jax.ShapeDtypeStruct((1, 1), f32),
        ],
        scratch_shapes=[
            pltpu.VMEM((_KP, _KP), f32),
            pltpu.SMEM((1, 1), f32),
        ],
    )(Srow, Scol, Z_pad, SS,
      m1_W1[:Din], m1_W1[Din:], m1_b1.reshape(1, 64),
      m1_W2, m1_b2.reshape(1, 1), lin1_W,
      m2_W1[:H], m2_W1[H:], m2_b1.reshape(1, 64),
      m2_W2, m2_b2.reshape(1, 1), lin2_W,
      fin_W, fin_b.reshape(1, Dout))

    return (out, mc[0, 0], ol[0, 0], Z_pad[:_KC], S32[:, :_KC])
